# baseline (device time: 41662 ns/iter reference)
import jax
import jax.numpy as jnp
from jax import lax
from jax.experimental import pallas as pl
from jax.experimental.pallas import tpu as pltpu

N_DEV = 16
B = 64
D = 1024
H = 2048
ROWS = B // N_DEV
C = 2
CW = D // C


def kernel(x, Win0, Wout0, Win1, Wout1, Win2, Wout2):
    def body(
        x_ref, win0_ref, wout0_ref, win1_ref, wout1_ref, win2_ref, wout2_ref,
        out_ref,
        wbuf_ref, obuf_ref, xbuf_ref, p_ref, r_ref, rs_ref,
        wsems, osems, local_sems,
        rs_send_sems, rs_recv_sems, ag_send_sems, ag_recv_sems,
    ):
        my_i = lax.axis_index("i")
        win_hbm = [win0_ref, win1_ref, win2_ref]
        wout_hbm = [wout0_ref, wout1_ref, wout2_ref]

        wcopies = [None, None, None]
        ocopies = [None, None, None]
        wcopies[0] = pltpu.make_async_copy(
            win_hbm[0], wbuf_ref.at[0], wsems.at[0]
        )
        wcopies[0].start()
        ocopies[0] = pltpu.make_async_copy(
            wout_hbm[0], obuf_ref.at[0], osems.at[0]
        )
        ocopies[0].start()

        barrier_sem = pltpu.get_barrier_semaphore()
        for d in range(1, N_DEV):
            tgt = (my_i + d) % N_DEV
            pl.semaphore_signal(
                barrier_sem, inc=1,
                device_id=(tgt,), device_id_type=pl.DeviceIdType.MESH,
            )
        pl.semaphore_wait(barrier_sem, N_DEV - 1)

        wcopies[0].wait()
        wcopies[1] = pltpu.make_async_copy(
            win_hbm[1], wbuf_ref.at[1], wsems.at[1]
        )
        wcopies[1].start()
        ocopies[1] = pltpu.make_async_copy(
            wout_hbm[1], obuf_ref.at[1], osems.at[1]
        )
        ocopies[1].start()
        h = jnp.maximum(
            jnp.dot(x_ref[...], wbuf_ref[0],
                    preferred_element_type=jnp.float32),
            0.0,
        )
        wcopies[2] = pltpu.make_async_copy(
            win_hbm[2], wbuf_ref.at[0], wsems.at[0]
        )
        wcopies[2].start()

        rs_sends = [[] for _ in range(C)]
        ag_sends = [[] for _ in range(C)]
        rs_own = [None] * C
        ag_own = [None] * C
        for k in range(3):
            ocopies[k].wait()
            for c in range(C):
                cs = pl.ds(c * CW, CW)
                pc = jnp.dot(h, obuf_ref[k % 2, :, cs],
                             preferred_element_type=jnp.float32)
                for t in range(N_DEV):
                    p_ref[t, :, cs] = pc[ROWS * t:ROWS * (t + 1), :]
                rs_sends[c] = []
                for d in range(1, N_DEV):
                    tgt = (my_i + d) % N_DEV
                    rdma = pltpu.make_async_remote_copy(
                        src_ref=p_ref.at[tgt, :, cs],
                        dst_ref=rs_ref.at[my_i, :, cs],
                        send_sem=rs_send_sems.at[c, tgt],
                        recv_sem=rs_recv_sems.at[c, my_i],
                        device_id=(tgt,),
                        device_id_type=pl.DeviceIdType.MESH,
                    )
                    rdma.start()
                    rs_sends[c].append(rdma)
                rs_own[c] = pltpu.make_async_copy(
                    p_ref.at[my_i, :, cs], rs_ref.at[my_i, :, cs],
                    local_sems.at[c],
                )
                rs_own[c].start()
            if k == 0:
                ocopies[2] = pltpu.make_async_copy(
                    wout_hbm[2], obuf_ref.at[0], osems.at[0]
                )
                ocopies[2].start()

            for c in range(C):
                cs = pl.ds(c * CW, CW)
                for d in range(1, N_DEV):
                    src = (my_i + d) % N_DEV
                    recv = pltpu.make_async_remote_copy(
                        src_ref=p_ref.at[0, :, cs],
                        dst_ref=rs_ref.at[src, :, cs],
                        send_sem=rs_send_sems.at[c, src],
                        recv_sem=rs_recv_sems.at[c, src],
                        device_id=(src,),
                        device_id_type=pl.DeviceIdType.MESH,
                    )
                    recv.wait_recv()
                rs_own[c].wait()
                rc = rs_ref[0, :, cs]
                for j in range(1, N_DEV):
                    rc = rc + rs_ref[j, :, cs]

                if k < 2:
                    r_ref[:, cs] = rc
                    ag_sends[c] = []
                    for d in range(1, N_DEV):
                        tgt = (my_i + d) % N_DEV
                        rdma = pltpu.make_async_remote_copy(
                            src_ref=r_ref.at[:, cs],
                            dst_ref=xbuf_ref.at[my_i, :, cs],
                            send_sem=ag_send_sems.at[c, tgt],
                            recv_sem=ag_recv_sems.at[c, my_i],
                            device_id=(tgt,),
                            device_id_type=pl.DeviceIdType.MESH,
                        )
                        rdma.start()
                        ag_sends[c].append(rdma)
                    ag_own[c] = pltpu.make_async_copy(
                        r_ref.at[:, cs], xbuf_ref.at[my_i, :, cs],
                        local_sems.at[C + c],
                    )
                    ag_own[c].start()
                else:
                    out_ref[:, cs] = rc

            if k < 2:
                wcopies[k + 1].wait()
                acc = None
                for c in range(C):
                    cs = pl.ds(c * CW, CW)
                    for d in range(1, N_DEV):
                        src = (my_i + d) % N_DEV
                        recv = pltpu.make_async_remote_copy(
                            src_ref=r_ref.at[:, cs],
                            dst_ref=xbuf_ref.at[src, :, cs],
                            send_sem=ag_send_sems.at[c, src],
                            recv_sem=ag_recv_sems.at[c, src],
                            device_id=(src,),
                            device_id_type=pl.DeviceIdType.MESH,
                        )
                        recv.wait_recv()
                    ag_own[c].wait()
                    xc = jnp.concatenate(
                        [xbuf_ref[j, :, cs] for j in range(N_DEV)], axis=0
                    )
                    part = jnp.dot(xc, wbuf_ref[(k + 1) % 2, cs, :],
                                   preferred_element_type=jnp.float32)
                    acc = part if acc is None else acc + part
                h = jnp.maximum(acc, 0.0)

            for c in range(C):
                for rdma in rs_sends[c]:
                    rdma.wait_send()
                if k < 2:
                    for rdma in ag_sends[c]:
                        rdma.wait_send()

    return pl.pallas_call(
        body,
        out_shape=jax.ShapeDtypeStruct((ROWS, D), jnp.float32),
        in_specs=[pl.BlockSpec(memory_space=pltpu.VMEM)]
        + [pl.BlockSpec(memory_space=pltpu.HBM)] * 6,
        out_specs=pl.BlockSpec(memory_space=pltpu.VMEM),
        scratch_shapes=[
            pltpu.VMEM((2, D, H), jnp.float32),
            pltpu.VMEM((2, H, D), jnp.float32),
            pltpu.VMEM((N_DEV, ROWS, D), jnp.float32),
            pltpu.VMEM((N_DEV, ROWS, D), jnp.float32),
            pltpu.VMEM((ROWS, D), jnp.float32),
            pltpu.VMEM((N_DEV, ROWS, D), jnp.float32),
            pltpu.SemaphoreType.DMA((2,)),
            pltpu.SemaphoreType.DMA((2,)),
            pltpu.SemaphoreType.DMA((2 * C,)),
            pltpu.SemaphoreType.DMA((C, N_DEV)),
            pltpu.SemaphoreType.DMA((C, N_DEV)),
            pltpu.SemaphoreType.DMA((C, N_DEV)),
            pltpu.SemaphoreType.DMA((C, N_DEV)),
        ],
        compiler_params=pltpu.CompilerParams(
            collective_id=0,
            vmem_limit_bytes=100 * 1024 * 1024,
        ),
    )(x, Win0, Wout0, Win1, Wout1, Win2, Wout2)
